# Initial kernel scaffold; baseline (speedup 1.0000x reference)
import jax
import jax.numpy as jnp
from jax.experimental import pallas as pl


def kernel(x, gate_w, gate_b, w1, b1, w2, b2):
    raise NotImplementedError("write your pallas kernel here")


# speedup vs baseline: 2.8730x; 2.8730x over previous
import functools

import jax
import jax.numpy as jnp
from jax.experimental import pallas as pl


_SQRT_HALF = 0.7071067811865476


def _router_kernel(x_ref, gw_ref, gb_ref, coef_ref, aux_ref, *, topk):
    n, e = coef_ref.shape
    logits = jnp.dot(x_ref[...], gw_ref[...],
                     preferred_element_type=jnp.float32) + gb_ref[...]
    m = jnp.max(logits, axis=-1, keepdims=True)
    ex = jnp.exp(logits - m)
    p = ex / jnp.sum(ex, axis=-1, keepdims=True)

    lane = jax.lax.broadcasted_iota(jnp.int32, (n, e), 1)
    v1 = jnp.max(p, axis=-1, keepdims=True)
    l1 = jnp.min(jnp.where(p == v1, lane, e), axis=-1, keepdims=True)
    sel1 = (lane == l1)
    p2 = jnp.where(sel1, -1.0, p)
    v2 = jnp.max(p2, axis=-1, keepdims=True)
    l2 = jnp.min(jnp.where(p2 == v2, lane, e), axis=-1, keepdims=True)
    sel2 = (lane == l2)

    den = v1 + v2
    coef = jnp.where(sel1, v1 / den, 0.0) + jnp.where(sel2, v2 / den, 0.0)
    coef_ref[...] = coef

    counts = jnp.sum(sel1.astype(jnp.float32) + sel2.astype(jnp.float32),
                     axis=0, keepdims=True)
    routing = jnp.mean(p, axis=0, keepdims=True)
    frac = counts / (n * topk)
    aux = e * jnp.sum(frac * routing)
    aux_ref[...] = jnp.broadcast_to(aux, (1, 1))


def _ffn_kernel(coef_ref, x_ref, w1_ref, b1_ref, w2_ref, b2_ref, out_ref):
    e = pl.program_id(1)
    f = pl.program_id(2)

    @pl.when((e == 0) & (f == 0))
    def _():
        out_ref[...] = jnp.zeros_like(out_ref)

    x = x_ref[...]
    h = jnp.dot(x, w1_ref[0], preferred_element_type=jnp.float32) + b1_ref[0]
    h = 0.5 * h * (1.0 + jax.lax.erf(h * _SQRT_HALF))
    y = jnp.dot(h, w2_ref[0], preferred_element_type=jnp.float32)
    y = y + jnp.where(f == 0, 1.0, 0.0) * b2_ref[0]

    lane = jax.lax.broadcasted_iota(jnp.int32, coef_ref.shape, 1)
    c = jnp.sum(jnp.where(lane == e, coef_ref[...], 0.0), axis=1,
                keepdims=True)
    out_ref[...] += c * y


def _router(xf, gate_w, gate_b, topk):
    n, d = xf.shape
    e = gate_w.shape[1]
    coef, aux = pl.pallas_call(
        functools.partial(_router_kernel, topk=topk),
        out_shape=(
            jax.ShapeDtypeStruct((n, e), jnp.float32),
            jax.ShapeDtypeStruct((1, 1), jnp.float32),
        ),
    )(xf, gate_w, gate_b.reshape(1, e))
    return coef, aux[0, 0]


def _ffn(xf, coef, w1, b1, w2, b2, bm=1024, bf=512):
    n, d = xf.shape
    e, _, ff = w1.shape
    grid = (n // bm, e, ff // bf)
    return pl.pallas_call(
        _ffn_kernel,
        grid=grid,
        in_specs=[
            pl.BlockSpec((bm, e), lambda m, ei, fi: (m, 0)),
            pl.BlockSpec((bm, d), lambda m, ei, fi: (m, 0)),
            pl.BlockSpec((1, d, bf), lambda m, ei, fi: (ei, 0, fi)),
            pl.BlockSpec((1, 1, bf), lambda m, ei, fi: (ei, 0, fi)),
            pl.BlockSpec((1, bf, d), lambda m, ei, fi: (ei, fi, 0)),
            pl.BlockSpec((1, 1, d), lambda m, ei, fi: (ei, 0, 0)),
        ],
        out_specs=pl.BlockSpec((bm, d), lambda m, ei, fi: (m, 0)),
        out_shape=jax.ShapeDtypeStruct((n, d), jnp.float32),
    )(coef, xf, w1, b1.reshape(e, 1, ff), w2, b2.reshape(e, 1, d))


def kernel(x, gate_w, gate_b, w1, b1, w2, b2):
    bq, sq, dq = x.shape
    xf = x.reshape(bq * sq, dq)
    coef, aux = _router(xf, gate_w, gate_b, topk=2)
    out = _ffn(xf, coef, w1, b1, w2, b2)
    return out.reshape(bq, sq, dq), aux


# speedup vs baseline: 2.9681x; 1.0331x over previous
import functools

import jax
import jax.numpy as jnp
from jax.experimental import pallas as pl


_SQRT_HALF = 0.7071067811865476


def _router_kernel(x_ref, gw_ref, gb_ref, coef_ref, aux_ref, *, topk):
    n, e = coef_ref.shape
    logits = jnp.dot(x_ref[...], gw_ref[...],
                     preferred_element_type=jnp.float32) + gb_ref[...]
    m = jnp.max(logits, axis=-1, keepdims=True)
    ex = jnp.exp(logits - m)
    p = ex / jnp.sum(ex, axis=-1, keepdims=True)

    lane = jax.lax.broadcasted_iota(jnp.int32, (n, e), 1)
    v1 = jnp.max(p, axis=-1, keepdims=True)
    l1 = jnp.min(jnp.where(p == v1, lane, e), axis=-1, keepdims=True)
    sel1 = (lane == l1)
    p2 = jnp.where(sel1, -1.0, p)
    v2 = jnp.max(p2, axis=-1, keepdims=True)
    l2 = jnp.min(jnp.where(p2 == v2, lane, e), axis=-1, keepdims=True)
    sel2 = (lane == l2)

    den = v1 + v2
    coef = jnp.where(sel1, v1 / den, 0.0) + jnp.where(sel2, v2 / den, 0.0)
    coef_ref[...] = coef

    counts = jnp.sum(sel1.astype(jnp.float32) + sel2.astype(jnp.float32),
                     axis=0, keepdims=True)
    routing = jnp.mean(p, axis=0, keepdims=True)
    frac = counts / (n * topk)
    aux = e * jnp.sum(frac * routing)
    aux_ref[...] = jnp.broadcast_to(aux, (1, 1))


def _ffn_kernel(coef_ref, x_ref, w1_ref, b1_ref, w2_ref, b2_ref, out_ref):
    e = pl.program_id(1)
    f = pl.program_id(2)

    @pl.when((e == 0) & (f == 0))
    def _():
        out_ref[...] = jnp.zeros_like(out_ref)

    x = x_ref[...].astype(jnp.bfloat16)
    h = jnp.dot(x, w1_ref[0].astype(jnp.bfloat16),
                preferred_element_type=jnp.float32) + b1_ref[0]
    h = 0.5 * h * (1.0 + jax.lax.erf(h * _SQRT_HALF))
    y = jnp.dot(h.astype(jnp.bfloat16), w2_ref[0].astype(jnp.bfloat16),
                preferred_element_type=jnp.float32)
    y = y + jnp.where(f == 0, 1.0, 0.0) * b2_ref[0]

    lane = jax.lax.broadcasted_iota(jnp.int32, coef_ref.shape, 1)
    c = jnp.sum(jnp.where(lane == e, coef_ref[...], 0.0), axis=1,
                keepdims=True)
    out_ref[...] += c * y


def _router(xf, gate_w, gate_b, topk):
    n, d = xf.shape
    e = gate_w.shape[1]
    coef, aux = pl.pallas_call(
        functools.partial(_router_kernel, topk=topk),
        out_shape=(
            jax.ShapeDtypeStruct((n, e), jnp.float32),
            jax.ShapeDtypeStruct((1, 1), jnp.float32),
        ),
    )(xf, gate_w, gate_b.reshape(1, e))
    return coef, aux[0, 0]


def _ffn(xf, coef, w1, b1, w2, b2, bm=2048, bf=512):
    n, d = xf.shape
    e, _, ff = w1.shape
    grid = (n // bm, e, ff // bf)
    return pl.pallas_call(
        _ffn_kernel,
        grid=grid,
        in_specs=[
            pl.BlockSpec((bm, e), lambda m, ei, fi: (m, 0)),
            pl.BlockSpec((bm, d), lambda m, ei, fi: (m, 0)),
            pl.BlockSpec((1, d, bf), lambda m, ei, fi: (ei, 0, fi)),
            pl.BlockSpec((1, 1, bf), lambda m, ei, fi: (ei, 0, fi)),
            pl.BlockSpec((1, bf, d), lambda m, ei, fi: (ei, fi, 0)),
            pl.BlockSpec((1, 1, d), lambda m, ei, fi: (ei, 0, 0)),
        ],
        out_specs=pl.BlockSpec((bm, d), lambda m, ei, fi: (m, 0)),
        out_shape=jax.ShapeDtypeStruct((n, d), jnp.float32),
    )(coef, xf, w1, b1.reshape(e, 1, ff), w2, b2.reshape(e, 1, d))


def kernel(x, gate_w, gate_b, w1, b1, w2, b2):
    bq, sq, dq = x.shape
    xf = x.reshape(bq * sq, dq)
    coef, aux = _router(xf, gate_w, gate_b, topk=2)
    out = _ffn(xf, coef, w1, b1, w2, b2)
    return out.reshape(bq, sq, dq), aux


# speedup vs baseline: 3.1668x; 1.0670x over previous
import functools

import jax
import jax.numpy as jnp
from jax.experimental import pallas as pl
from jax.experimental.pallas import tpu as pltpu


_SQRT_HALF = 0.7071067811865476


def _shift_down(a, d):
    return jnp.concatenate([jnp.zeros((d, a.shape[1]), a.dtype), a[:-d]], axis=0)


def _router_kernel(x_ref, gw_ref, gb_ref, posa_ref, posb_ref, wa_ref,
                   wb_ref, counts_ref, aux_ref, *, topk, bm):
    n = x_ref.shape[0]
    e = gw_ref.shape[1]
    logits = jnp.dot(x_ref[...], gw_ref[...],
                     preferred_element_type=jnp.float32) + gb_ref[...]
    m = jnp.max(logits, axis=-1, keepdims=True)
    ex = jnp.exp(logits - m)
    p = ex / jnp.sum(ex, axis=-1, keepdims=True)

    lane = jax.lax.broadcasted_iota(jnp.int32, (n, e), 1)
    v1 = jnp.max(p, axis=-1, keepdims=True)
    l1 = jnp.min(jnp.where(p == v1, lane, e), axis=-1, keepdims=True)
    sel1 = (lane == l1)
    p2 = jnp.where(sel1, -1.0, p)
    v2 = jnp.max(p2, axis=-1, keepdims=True)
    l2 = jnp.min(jnp.where(p2 == v2, lane, e), axis=-1, keepdims=True)
    sel2 = (lane == l2)

    den = v1 + v2
    wa_ref[...] = v1 / den
    wb_ref[...] = v2 / den

    c1 = sel1.astype(jnp.float32)
    c2 = sel2.astype(jnp.float32)
    d = 1
    while d < n:
        c1 = c1 + _shift_down(c1, d)
        c2 = c2 + _shift_down(c2, d)
        d *= 2
    tot1 = c1[n - 1:n, :]
    tot2 = c2[n - 1:n, :]
    counts = tot1 + tot2
    counts_ref[...] = counts

    pc = jnp.ceil(counts / bm) * bm
    inc = pc
    d = 1
    while d < e:
        inc = inc + jnp.concatenate(
            [jnp.zeros((1, d), jnp.float32), inc[:, :-d]], axis=1)
        d *= 2
    base = inc - pc

    posa_f = base + c1 - 1.0
    posb_f = base + tot1 + c2 - 1.0
    posa_ref[...] = jnp.sum(
        jnp.where(sel1, posa_f, 0.0), axis=1, keepdims=True).astype(jnp.int32)
    posb_ref[...] = jnp.sum(
        jnp.where(sel2, posb_f, 0.0), axis=1, keepdims=True).astype(jnp.int32)

    routing = jnp.mean(p, axis=0, keepdims=True)
    frac = counts / (n * topk)
    aux = e * jnp.sum(frac * routing)
    aux_ref[...] = jnp.broadcast_to(aux, (1, 1))


def _router(xf, gate_w, gate_b, topk, bm):
    n, d = xf.shape
    e = gate_w.shape[1]
    outs = pl.pallas_call(
        functools.partial(_router_kernel, topk=topk, bm=bm),
        out_shape=(
            jax.ShapeDtypeStruct((n, 1), jnp.int32),
            jax.ShapeDtypeStruct((n, 1), jnp.int32),
            jax.ShapeDtypeStruct((n, 1), jnp.float32),
            jax.ShapeDtypeStruct((n, 1), jnp.float32),
            jax.ShapeDtypeStruct((1, e), jnp.float32),
            jax.ShapeDtypeStruct((1, 1), jnp.float32),
        ),
    )(xf, gate_w, gate_b.reshape(1, e))
    posa, posb, wa, wb, counts, aux = outs
    return posa, posb, wa, wb, counts, aux[0, 0]


def _gffn_kernel(gid_ref, act_ref, xs_ref, w1_ref, b1_ref, w2_ref, b2_ref,
                 out_ref):
    t = pl.program_id(0)
    f = pl.program_id(1)
    active = act_ref[t] == 1

    @pl.when(active)
    def _():
        x = xs_ref[...].astype(jnp.bfloat16)
        h = jnp.dot(x, w1_ref[0].astype(jnp.bfloat16),
                    preferred_element_type=jnp.float32) + b1_ref[0]
        h = 0.5 * h * (1.0 + jax.lax.erf(h * _SQRT_HALF))
        y = jnp.dot(h.astype(jnp.bfloat16), w2_ref[0].astype(jnp.bfloat16),
                    preferred_element_type=jnp.float32)

        @pl.when(f == 0)
        def _():
            out_ref[...] = y + b2_ref[0]

        @pl.when(f != 0)
        def _():
            out_ref[...] += y


def _gffn(xs, gid, act, w1, b1, w2, b2, bm, bf):
    npad = xs.shape[0]
    e, d, ff = w1.shape
    t_tiles = npad // bm
    nf = ff // bf
    grid_spec = pltpu.PrefetchScalarGridSpec(
        num_scalar_prefetch=2,
        grid=(t_tiles, nf),
        in_specs=[
            pl.BlockSpec((bm, d), lambda t, f, gid, act: (t, 0)),
            pl.BlockSpec((1, d, bf), lambda t, f, gid, act: (gid[t], 0, f)),
            pl.BlockSpec((1, 1, bf), lambda t, f, gid, act: (gid[t], 0, f)),
            pl.BlockSpec((1, bf, d), lambda t, f, gid, act: (gid[t], f, 0)),
            pl.BlockSpec((1, 1, d), lambda t, f, gid, act: (gid[t], 0, 0)),
        ],
        out_specs=pl.BlockSpec((bm, d), lambda t, f, gid, act: (t, 0)),
    )
    return pl.pallas_call(
        _gffn_kernel,
        grid_spec=grid_spec,
        out_shape=jax.ShapeDtypeStruct((npad, d), jnp.float32),
    )(gid, act, xs, w1, b1.reshape(e, 1, ff), w2, b2.reshape(e, 1, d))


def kernel(x, gate_w, gate_b, w1, b1, w2, b2):
    bq, sq, dq = x.shape
    n = bq * sq
    e, _, ff = w1.shape
    topk = 2
    bm = 512
    bf = 512
    npad = ((n * topk + e * (bm - 1)) // bm + 1) * bm

    xf = x.reshape(n, dq)
    posa, posb, wa, wb, counts, aux = _router(xf, gate_w, gate_b, topk, bm)
    pa = posa.reshape(n)
    pb = posb.reshape(n)

    ci = counts.reshape(e).astype(jnp.int32)
    pc = ((ci + bm - 1) // bm) * bm
    ends = jnp.cumsum(pc)
    tile_start = jnp.arange(npad // bm, dtype=jnp.int32) * bm
    gid = jnp.sum((tile_start[:, None] >= ends[None, :]).astype(jnp.int32),
                  axis=1).astype(jnp.int32)
    act = (tile_start < ends[-1]).astype(jnp.int32)

    xs = jnp.zeros((npad, dq), jnp.float32).at[pa].set(xf).at[pb].set(xf)

    ys = _gffn(xs, gid, act, w1, b1, w2, b2, bm, bf)

    out = wa * ys[pa] + wb * ys[pb]
    return out.reshape(bq, sq, dq), aux


# speedup vs baseline: 4.2175x; 1.3318x over previous
import functools

import jax
import jax.numpy as jnp
from jax.experimental import pallas as pl
from jax.experimental.pallas import tpu as pltpu


_SQRT_HALF = 0.7071067811865476


def _shift_down(a, d):
    return jnp.concatenate([jnp.zeros((d, a.shape[1]), a.dtype), a[:-d]], axis=0)


def _router_kernel(x_ref, gw_ref, gb_ref, posa_ref, posb_ref, wa_ref,
                   wb_ref, counts_ref, aux_ref, *, topk, bm):
    n = x_ref.shape[0]
    e = gw_ref.shape[1]
    logits = jnp.dot(x_ref[...], gw_ref[...],
                     preferred_element_type=jnp.float32) + gb_ref[...]
    m = jnp.max(logits, axis=-1, keepdims=True)
    ex = jnp.exp(logits - m)
    p = ex / jnp.sum(ex, axis=-1, keepdims=True)

    lane = jax.lax.broadcasted_iota(jnp.int32, (n, e), 1)
    v1 = jnp.max(p, axis=-1, keepdims=True)
    l1 = jnp.min(jnp.where(p == v1, lane, e), axis=-1, keepdims=True)
    sel1 = (lane == l1)
    p2 = jnp.where(sel1, -1.0, p)
    v2 = jnp.max(p2, axis=-1, keepdims=True)
    l2 = jnp.min(jnp.where(p2 == v2, lane, e), axis=-1, keepdims=True)
    sel2 = (lane == l2)

    den = v1 + v2
    wa_ref[...] = v1 / den
    wb_ref[...] = v2 / den

    c1 = sel1.astype(jnp.float32)
    c2 = sel2.astype(jnp.float32)
    d = 1
    while d < n:
        c1 = c1 + _shift_down(c1, d)
        c2 = c2 + _shift_down(c2, d)
        d *= 2
    tot1 = c1[n - 1:n, :]
    tot2 = c2[n - 1:n, :]
    counts = tot1 + tot2
    counts_ref[...] = counts

    pc = jnp.ceil(counts / bm) * bm
    inc = pc
    d = 1
    while d < e:
        inc = inc + jnp.concatenate(
            [jnp.zeros((1, d), jnp.float32), inc[:, :-d]], axis=1)
        d *= 2
    base = inc - pc

    posa_f = base + c1 - 1.0
    posb_f = base + tot1 + c2 - 1.0
    posa_ref[...] = jnp.sum(
        jnp.where(sel1, posa_f, 0.0), axis=1, keepdims=True).astype(jnp.int32)
    posb_ref[...] = jnp.sum(
        jnp.where(sel2, posb_f, 0.0), axis=1, keepdims=True).astype(jnp.int32)

    routing = jnp.mean(p, axis=0, keepdims=True)
    frac = counts / (n * topk)
    aux = e * jnp.sum(frac * routing)
    aux_ref[...] = jnp.broadcast_to(aux, (1, 1))


def _router(xf, gate_w, gate_b, topk, bm):
    n, d = xf.shape
    e = gate_w.shape[1]
    outs = pl.pallas_call(
        functools.partial(_router_kernel, topk=topk, bm=bm),
        out_shape=(
            jax.ShapeDtypeStruct((n, 1), jnp.int32),
            jax.ShapeDtypeStruct((n, 1), jnp.int32),
            jax.ShapeDtypeStruct((n, 1), jnp.float32),
            jax.ShapeDtypeStruct((n, 1), jnp.float32),
            jax.ShapeDtypeStruct((1, e), jnp.float32),
            jax.ShapeDtypeStruct((1, 1), jnp.float32),
        ),
    )(xf, gate_w, gate_b.reshape(1, e))
    posa, posb, wa, wb, counts, aux = outs
    return posa, posb, wa, wb, counts, aux[0, 0]


def _gffn_kernel(gid_ref, act_ref, xs_ref, w1_ref, b1_ref, w2_ref, b2_ref,
                 out_ref):
    t = pl.program_id(0)
    active = act_ref[t] == 1

    @pl.when(active)
    def _():
        x = xs_ref[...].astype(jnp.bfloat16)
        h = jnp.dot(x, w1_ref[0].astype(jnp.bfloat16),
                    preferred_element_type=jnp.float32) + b1_ref[0]
        h = 0.5 * h * (1.0 + jax.lax.erf(h * _SQRT_HALF))
        y = jnp.dot(h.astype(jnp.bfloat16), w2_ref[0].astype(jnp.bfloat16),
                    preferred_element_type=jnp.float32)
        out_ref[...] = y + b2_ref[0]


def _gffn(xs, gid, act, w1, b1, w2, b2, bm):
    npad = xs.shape[0]
    e, d, ff = w1.shape
    t_tiles = npad // bm
    grid_spec = pltpu.PrefetchScalarGridSpec(
        num_scalar_prefetch=2,
        grid=(t_tiles,),
        in_specs=[
            pl.BlockSpec((bm, d), lambda t, gid, act: (t, 0)),
            pl.BlockSpec((1, d, ff), lambda t, gid, act: (gid[t], 0, 0)),
            pl.BlockSpec((1, 1, ff), lambda t, gid, act: (gid[t], 0, 0)),
            pl.BlockSpec((1, ff, d), lambda t, gid, act: (gid[t], 0, 0)),
            pl.BlockSpec((1, 1, d), lambda t, gid, act: (gid[t], 0, 0)),
        ],
        out_specs=pl.BlockSpec((bm, d), lambda t, gid, act: (t, 0)),
    )
    return pl.pallas_call(
        _gffn_kernel,
        grid_spec=grid_spec,
        out_shape=jax.ShapeDtypeStruct((npad, d), jnp.float32),
    )(gid, act, xs, w1, b1.reshape(e, 1, ff), w2, b2.reshape(e, 1, d))


def kernel(x, gate_w, gate_b, w1, b1, w2, b2):
    bq, sq, dq = x.shape
    n = bq * sq
    e, _, ff = w1.shape
    topk = 2
    bm = 512
    npad = ((n * topk + e * (bm - 1)) // bm + 1) * bm

    xf = x.reshape(n, dq)
    posa, posb, wa, wb, counts, aux = _router(xf, gate_w, gate_b, topk, bm)
    pa = posa.reshape(n)
    pb = posb.reshape(n)

    ci = counts.reshape(e).astype(jnp.int32)
    pc = ((ci + bm - 1) // bm) * bm
    ends = jnp.cumsum(pc)
    tile_start = jnp.arange(npad // bm, dtype=jnp.int32) * bm
    gid = jnp.sum((tile_start[:, None] >= ends[None, :]).astype(jnp.int32),
                  axis=1).astype(jnp.int32)
    act = (tile_start < ends[-1]).astype(jnp.int32)

    xs = jnp.zeros((npad, dq), jnp.float32).at[pa].set(xf).at[pb].set(xf)

    ys = _gffn(xs, gid, act, w1, b1, w2, b2, bm)

    out = wa * ys[pa] + wb * ys[pb]
    return out.reshape(bq, sq, dq), aux


# speedup vs baseline: 5.5197x; 1.3087x over previous
import functools

import jax
import jax.numpy as jnp
from jax import lax
from jax.experimental import pallas as pl
from jax.experimental.pallas import tpu as pltpu
from jax.experimental.pallas import tpu_sc as plsc


_SQRT_HALF = 0.7071067811865476


def _shift_down(a, d):
    return jnp.concatenate([jnp.zeros((d, a.shape[1]), a.dtype), a[:-d]], axis=0)


def _router_kernel(x_ref, gw_ref, gb_ref, posa_ref, posb_ref, wa_ref,
                   wb_ref, counts_ref, aux_ref, *, topk, bm):
    n = x_ref.shape[0]
    e = gw_ref.shape[1]
    logits = jnp.dot(x_ref[...], gw_ref[...],
                     preferred_element_type=jnp.float32) + gb_ref[...]
    m = jnp.max(logits, axis=-1, keepdims=True)
    ex = jnp.exp(logits - m)
    p = ex / jnp.sum(ex, axis=-1, keepdims=True)

    lane = jax.lax.broadcasted_iota(jnp.int32, (n, e), 1)
    v1 = jnp.max(p, axis=-1, keepdims=True)
    l1 = jnp.min(jnp.where(p == v1, lane, e), axis=-1, keepdims=True)
    sel1 = (lane == l1)
    p2 = jnp.where(sel1, -1.0, p)
    v2 = jnp.max(p2, axis=-1, keepdims=True)
    l2 = jnp.min(jnp.where(p2 == v2, lane, e), axis=-1, keepdims=True)
    sel2 = (lane == l2)

    den = v1 + v2
    wa_ref[...] = jnp.broadcast_to(v1 / den, wa_ref.shape)
    wb_ref[...] = jnp.broadcast_to(v2 / den, wb_ref.shape)

    c1 = sel1.astype(jnp.float32)
    c2 = sel2.astype(jnp.float32)
    d = 1
    while d < n:
        c1 = c1 + _shift_down(c1, d)
        c2 = c2 + _shift_down(c2, d)
        d *= 2
    tot1 = c1[n - 1:n, :]
    tot2 = c2[n - 1:n, :]
    counts = tot1 + tot2
    counts_ref[...] = counts

    pc = jnp.ceil(counts / bm) * bm
    inc = pc
    d = 1
    while d < e:
        inc = inc + jnp.concatenate(
            [jnp.zeros((1, d), jnp.float32), inc[:, :-d]], axis=1)
        d *= 2
    base = inc - pc

    posa_f = base + c1 - 1.0
    posb_f = base + tot1 + c2 - 1.0
    posa_ref[...] = jnp.sum(
        jnp.where(sel1, posa_f, 0.0), axis=1, keepdims=True).astype(jnp.int32)
    posb_ref[...] = jnp.sum(
        jnp.where(sel2, posb_f, 0.0), axis=1, keepdims=True).astype(jnp.int32)

    routing = jnp.mean(p, axis=0, keepdims=True)
    frac = counts / (n * topk)
    aux = e * jnp.sum(frac * routing)
    aux_ref[...] = jnp.broadcast_to(aux, (1, 1))


def _router(xf, gate_w, gate_b, topk, bm):
    n, d = xf.shape
    e = gate_w.shape[1]
    outs = pl.pallas_call(
        functools.partial(_router_kernel, topk=topk, bm=bm),
        out_shape=(
            jax.ShapeDtypeStruct((n, 1), jnp.int32),
            jax.ShapeDtypeStruct((n, 1), jnp.int32),
            jax.ShapeDtypeStruct((n, 128), jnp.float32),
            jax.ShapeDtypeStruct((n, 128), jnp.float32),
            jax.ShapeDtypeStruct((1, e), jnp.float32),
            jax.ShapeDtypeStruct((1, 1), jnp.float32),
        ),
    )(xf, gate_w, gate_b.reshape(1, e))
    posa, posb, wa, wb, counts, aux = outs
    return posa, posb, wa, wb, counts, aux[0, 0]


def _gffn_kernel(gid_ref, act_ref, xs_ref, ws_ref, w1_ref, b1_ref, w2_ref,
                 b2_ref, out_ref):
    t = pl.program_id(0)
    active = act_ref[t] == 1

    @pl.when(active)
    def _():
        x = xs_ref[...].astype(jnp.bfloat16)
        h = jnp.dot(x, w1_ref[0].astype(jnp.bfloat16),
                    preferred_element_type=jnp.float32) + b1_ref[0]
        h = 0.5 * h * (1.0 + jax.lax.erf(h * _SQRT_HALF))
        y = jnp.dot(h.astype(jnp.bfloat16), w2_ref[0].astype(jnp.bfloat16),
                    preferred_element_type=jnp.float32)
        out_ref[...] = (y + b2_ref[0]) * ws_ref[:, :1]


def _gffn(xs, ws, gid, act, w1, b1, w2, b2, bm):
    npad = xs.shape[0]
    e, d, ff = w1.shape
    t_tiles = npad // bm
    grid_spec = pltpu.PrefetchScalarGridSpec(
        num_scalar_prefetch=2,
        grid=(t_tiles,),
        in_specs=[
            pl.BlockSpec((bm, d), lambda t, gid, act: (t, 0)),
            pl.BlockSpec((bm, 128), lambda t, gid, act: (t, 0)),
            pl.BlockSpec((1, d, ff), lambda t, gid, act: (gid[t], 0, 0)),
            pl.BlockSpec((1, 1, ff), lambda t, gid, act: (gid[t], 0, 0)),
            pl.BlockSpec((1, ff, d), lambda t, gid, act: (gid[t], 0, 0)),
            pl.BlockSpec((1, 1, d), lambda t, gid, act: (gid[t], 0, 0)),
        ],
        out_specs=pl.BlockSpec((bm, d), lambda t, gid, act: (t, 0)),
    )
    return pl.pallas_call(
        _gffn_kernel,
        grid_spec=grid_spec,
        out_shape=jax.ShapeDtypeStruct((npad, d), jnp.float32),
    )(gid, act, xs, ws, w1, b1.reshape(e, 1, ff), w2, b2.reshape(e, 1, d))


def _dispatch_sc(xf, pa, pb, wa, wb, npad):
    n, d = xf.shape
    info = plsc.get_sparse_core_info()
    nw = info.num_cores * info.num_subcores
    per_w = n // nw
    mesh = plsc.VectorSubcoreMesh(core_axis_name="c", subcore_axis_name="s")

    @functools.partial(
        pl.kernel, mesh=mesh,
        out_type=(
            jax.ShapeDtypeStruct((npad, d), jnp.float32),
            jax.ShapeDtypeStruct((npad, 128), jnp.float32),
        ),
        scratch_types=[
            pltpu.VMEM((64,), jnp.int32),
            pltpu.VMEM((64,), jnp.int32),
            pltpu.VMEM((64, d), jnp.float32),
            pltpu.VMEM((64, 128), jnp.float32),
            pltpu.VMEM((64, 128), jnp.float32),
            pltpu.SemaphoreType.DMA,
        ],
    )
    def k(x_hbm, pa_hbm, pb_hbm, wa_hbm, wb_hbm, xs_hbm, ws_hbm,
          ia_v, ib_v, rows_v, wav_v, wbv_v, sem):
        wid = lax.axis_index("s") * info.num_cores + lax.axis_index("c")
        for blk in range(per_w // 64):
            base = wid * per_w + blk * 64
            pltpu.sync_copy(x_hbm.at[pl.ds(base, 64)], rows_v)
            pltpu.sync_copy(pa_hbm.at[pl.ds(base, 64)], ia_v)
            pltpu.sync_copy(pb_hbm.at[pl.ds(base, 64)], ib_v)
            pltpu.sync_copy(wa_hbm.at[pl.ds(base, 64)], wav_v)
            pltpu.sync_copy(wb_hbm.at[pl.ds(base, 64)], wbv_v)
            pltpu.async_copy(rows_v, xs_hbm.at[ia_v], sem).wait()
            pltpu.async_copy(rows_v, xs_hbm.at[ib_v], sem).wait()
            pltpu.async_copy(wav_v, ws_hbm.at[ia_v], sem).wait()
            pltpu.async_copy(wbv_v, ws_hbm.at[ib_v], sem).wait()

    return k(xf, pa, pb, wa, wb)


def _combine_sc(ys, pa, pb, n, d):
    info = plsc.get_sparse_core_info()
    nw = info.num_cores * info.num_subcores
    per_w = n // nw
    sb = 64
    nlane = info.num_lanes
    mesh = plsc.VectorSubcoreMesh(core_axis_name="c", subcore_axis_name="s")

    @functools.partial(
        pl.kernel, mesh=mesh,
        out_type=jax.ShapeDtypeStruct((n, d), jnp.float32),
        scratch_types=[
            pltpu.VMEM((sb,), jnp.int32),
            pltpu.VMEM((sb,), jnp.int32),
            pltpu.VMEM((sb, d), jnp.float32),
            pltpu.VMEM((sb, d), jnp.float32),
            pltpu.SemaphoreType.DMA,
        ],
    )
    def k(ys_hbm, pa_hbm, pb_hbm, out_hbm, ia_v, ib_v, ra_v, rb_v, sem):
        wid = lax.axis_index("s") * info.num_cores + lax.axis_index("c")
        base = wid * per_w
        for blk in range(per_w // sb):
            off = base + blk * sb
            pltpu.sync_copy(pa_hbm.at[pl.ds(off, sb)], ia_v)
            pltpu.sync_copy(pb_hbm.at[pl.ds(off, sb)], ib_v)
            pltpu.async_copy(ys_hbm.at[ia_v], ra_v, sem).wait()
            pltpu.async_copy(ys_hbm.at[ib_v], rb_v, sem).wait()

            def body(i, _):
                for j in range(d // nlane):
                    sl = pl.ds(j * nlane, nlane)
                    ra_v[i, sl] = ra_v[i, sl] + rb_v[i, sl]
                return 0

            lax.fori_loop(0, sb, body, 0)
            pltpu.sync_copy(ra_v, out_hbm.at[pl.ds(off, sb)])

    return k(ys, pa, pb)


def kernel(x, gate_w, gate_b, w1, b1, w2, b2):
    bq, sq, dq = x.shape
    n = bq * sq
    e, _, ff = w1.shape
    topk = 2
    bm = 512
    npad = ((n * topk + e * (bm - 1)) // bm + 1) * bm

    xf = x.reshape(n, dq)
    posa, posb, wa, wb, counts, aux = _router(xf, gate_w, gate_b, topk, bm)
    pa = posa.reshape(n)
    pb = posb.reshape(n)

    ci = counts.reshape(e).astype(jnp.int32)
    pc = ((ci + bm - 1) // bm) * bm
    ends = jnp.cumsum(pc)
    tile_start = jnp.arange(npad // bm, dtype=jnp.int32) * bm
    gid = jnp.sum((tile_start[:, None] >= ends[None, :]).astype(jnp.int32),
                  axis=1).astype(jnp.int32)
    act = (tile_start < ends[-1]).astype(jnp.int32)

    xs, ws = _dispatch_sc(xf, pa, pb, wa, wb, npad)

    ys = _gffn(xs, ws, gid, act, w1, b1, w2, b2, bm)

    out = _combine_sc(ys, pa, pb, n, dq)
    return out.reshape(bq, sq, dq), aux


# speedup vs baseline: 5.5535x; 1.0061x over previous
import functools

import jax
import jax.numpy as jnp
from jax import lax
from jax.experimental import pallas as pl
from jax.experimental.pallas import tpu as pltpu
from jax.experimental.pallas import tpu_sc as plsc


_SQRT_HALF = 0.7071067811865476


def _shift_down(a, d):
    return jnp.concatenate([jnp.zeros((d, a.shape[1]), a.dtype), a[:-d]], axis=0)


def _router_kernel(x_ref, gw_ref, gb_ref, posa_ref, posb_ref, wa_ref,
                   wb_ref, counts_ref, aux_ref, *, topk, bm):
    n = x_ref.shape[0]
    e = gw_ref.shape[1]
    logits = jnp.dot(x_ref[...], gw_ref[...],
                     preferred_element_type=jnp.float32) + gb_ref[...]
    m = jnp.max(logits, axis=-1, keepdims=True)
    ex = jnp.exp(logits - m)
    p = ex / jnp.sum(ex, axis=-1, keepdims=True)

    lane = jax.lax.broadcasted_iota(jnp.int32, (n, e), 1)
    v1 = jnp.max(p, axis=-1, keepdims=True)
    l1 = jnp.min(jnp.where(p == v1, lane, e), axis=-1, keepdims=True)
    sel1 = (lane == l1)
    p2 = jnp.where(sel1, -1.0, p)
    v2 = jnp.max(p2, axis=-1, keepdims=True)
    l2 = jnp.min(jnp.where(p2 == v2, lane, e), axis=-1, keepdims=True)
    sel2 = (lane == l2)

    den = v1 + v2
    wa_ref[...] = jnp.broadcast_to(v1 / den, wa_ref.shape)
    wb_ref[...] = jnp.broadcast_to(v2 / den, wb_ref.shape)

    c1 = sel1.astype(jnp.float32)
    c2 = sel2.astype(jnp.float32)
    d = 1
    while d < n:
        c1 = c1 + _shift_down(c1, d)
        c2 = c2 + _shift_down(c2, d)
        d *= 2
    tot1 = c1[n - 1:n, :]
    tot2 = c2[n - 1:n, :]
    counts = tot1 + tot2
    counts_ref[...] = counts

    pc = jnp.ceil(counts / bm) * bm
    inc = pc
    d = 1
    while d < e:
        inc = inc + jnp.concatenate(
            [jnp.zeros((1, d), jnp.float32), inc[:, :-d]], axis=1)
        d *= 2
    base = inc - pc

    posa_f = base + c1 - 1.0
    posb_f = base + tot1 + c2 - 1.0
    posa_ref[...] = jnp.sum(
        jnp.where(sel1, posa_f, 0.0), axis=1, keepdims=True).astype(jnp.int32)
    posb_ref[...] = jnp.sum(
        jnp.where(sel2, posb_f, 0.0), axis=1, keepdims=True).astype(jnp.int32)

    routing = jnp.mean(p, axis=0, keepdims=True)
    frac = counts / (n * topk)
    aux = e * jnp.sum(frac * routing)
    aux_ref[...] = jnp.broadcast_to(aux, (1, 1))


def _router(xf, gate_w, gate_b, topk, bm):
    n, d = xf.shape
    e = gate_w.shape[1]
    outs = pl.pallas_call(
        functools.partial(_router_kernel, topk=topk, bm=bm),
        out_shape=(
            jax.ShapeDtypeStruct((n, 1), jnp.int32),
            jax.ShapeDtypeStruct((n, 1), jnp.int32),
            jax.ShapeDtypeStruct((n, 128), jnp.float32),
            jax.ShapeDtypeStruct((n, 128), jnp.float32),
            jax.ShapeDtypeStruct((1, e), jnp.float32),
            jax.ShapeDtypeStruct((1, 1), jnp.float32),
        ),
    )(xf, gate_w, gate_b.reshape(1, e))
    posa, posb, wa, wb, counts, aux = outs
    return posa, posb, wa, wb, counts, aux[0, 0]


def _gffn_kernel(gid_ref, act_ref, xs_ref, ws_ref, w1_ref, b1_ref, w2_ref,
                 b2_ref, out_ref):
    t = pl.program_id(0)
    active = act_ref[t] == 1

    @pl.when(active)
    def _():
        x = xs_ref[...].astype(jnp.bfloat16)
        h = jnp.dot(x, w1_ref[0].astype(jnp.bfloat16),
                    preferred_element_type=jnp.float32) + b1_ref[0]
        h = 0.5 * h * (1.0 + jax.lax.erf(h * _SQRT_HALF))
        y = jnp.dot(h.astype(jnp.bfloat16), w2_ref[0].astype(jnp.bfloat16),
                    preferred_element_type=jnp.float32)
        out_ref[...] = (y + b2_ref[0]) * ws_ref[:, :1]


def _gffn(xs, ws, gid, act, w1, b1, w2, b2, bm):
    npad = xs.shape[0]
    e, d, ff = w1.shape
    t_tiles = npad // bm
    grid_spec = pltpu.PrefetchScalarGridSpec(
        num_scalar_prefetch=2,
        grid=(t_tiles,),
        in_specs=[
            pl.BlockSpec((bm, d), lambda t, gid, act: (t, 0)),
            pl.BlockSpec((bm, 128), lambda t, gid, act: (t, 0)),
            pl.BlockSpec((1, d, ff), lambda t, gid, act: (gid[t], 0, 0)),
            pl.BlockSpec((1, 1, ff), lambda t, gid, act: (gid[t], 0, 0)),
            pl.BlockSpec((1, ff, d), lambda t, gid, act: (gid[t], 0, 0)),
            pl.BlockSpec((1, 1, d), lambda t, gid, act: (gid[t], 0, 0)),
        ],
        out_specs=pl.BlockSpec((bm, d), lambda t, gid, act: (t, 0)),
    )
    return pl.pallas_call(
        _gffn_kernel,
        grid_spec=grid_spec,
        out_shape=jax.ShapeDtypeStruct((npad, d), jnp.float32),
    )(gid, act, xs, ws, w1, b1.reshape(e, 1, ff), w2, b2.reshape(e, 1, d))


def _dispatch_sc(xf, pa, pb, wa, wb, npad):
    n, d = xf.shape
    info = plsc.get_sparse_core_info()
    nw = info.num_cores * info.num_subcores
    per_w = n // nw
    mesh = plsc.VectorSubcoreMesh(core_axis_name="c", subcore_axis_name="s")

    @functools.partial(
        pl.kernel, mesh=mesh,
        out_type=(
            jax.ShapeDtypeStruct((npad, d), jnp.float32),
            jax.ShapeDtypeStruct((npad, 128), jnp.float32),
        ),
        scratch_types=[
            pltpu.VMEM((64,), jnp.int32),
            pltpu.VMEM((64,), jnp.int32),
            pltpu.VMEM((64, d), jnp.float32),
            pltpu.VMEM((64, 128), jnp.float32),
            pltpu.VMEM((64, 128), jnp.float32),
            pltpu.SemaphoreType.DMA,
        ],
    )
    def k(x_hbm, pa_hbm, pb_hbm, wa_hbm, wb_hbm, xs_hbm, ws_hbm,
          ia_v, ib_v, rows_v, wav_v, wbv_v, sem):
        wid = lax.axis_index("s") * info.num_cores + lax.axis_index("c")
        for blk in range(per_w // 64):
            base = wid * per_w + blk * 64
            pltpu.sync_copy(x_hbm.at[pl.ds(base, 64)], rows_v)
            pltpu.sync_copy(pa_hbm.at[pl.ds(base, 64)], ia_v)
            pltpu.sync_copy(pb_hbm.at[pl.ds(base, 64)], ib_v)
            pltpu.sync_copy(wa_hbm.at[pl.ds(base, 64)], wav_v)
            pltpu.sync_copy(wb_hbm.at[pl.ds(base, 64)], wbv_v)
            c1 = pltpu.async_copy(rows_v, xs_hbm.at[ia_v], sem)
            c2 = pltpu.async_copy(rows_v, xs_hbm.at[ib_v], sem)
            c3 = pltpu.async_copy(wav_v, ws_hbm.at[ia_v], sem)
            c4 = pltpu.async_copy(wbv_v, ws_hbm.at[ib_v], sem)
            c1.wait(); c2.wait(); c3.wait(); c4.wait()

    return k(xf, pa, pb, wa, wb)


def _combine_sc(ys, pa, pb, n, d):
    info = plsc.get_sparse_core_info()
    nw = info.num_cores * info.num_subcores
    per_w = n // nw
    sb = 64
    nlane = info.num_lanes
    mesh = plsc.VectorSubcoreMesh(core_axis_name="c", subcore_axis_name="s")

    @functools.partial(
        pl.kernel, mesh=mesh,
        out_type=jax.ShapeDtypeStruct((n, d), jnp.float32),
        scratch_types=[
            pltpu.VMEM((sb,), jnp.int32),
            pltpu.VMEM((sb,), jnp.int32),
            pltpu.VMEM((sb, d), jnp.float32),
            pltpu.VMEM((sb, d), jnp.float32),
            pltpu.SemaphoreType.DMA,
        ],
    )
    def k(ys_hbm, pa_hbm, pb_hbm, out_hbm, ia_v, ib_v, ra_v, rb_v, sem):
        wid = lax.axis_index("s") * info.num_cores + lax.axis_index("c")
        base = wid * per_w
        for blk in range(per_w // sb):
            off = base + blk * sb
            pltpu.sync_copy(pa_hbm.at[pl.ds(off, sb)], ia_v)
            pltpu.sync_copy(pb_hbm.at[pl.ds(off, sb)], ib_v)
            g1 = pltpu.async_copy(ys_hbm.at[ia_v], ra_v, sem)
            g2 = pltpu.async_copy(ys_hbm.at[ib_v], rb_v, sem)
            g1.wait(); g2.wait()

            def body(i, _):
                for j in range(d // nlane):
                    sl = pl.ds(j * nlane, nlane)
                    ra_v[i, sl] = ra_v[i, sl] + rb_v[i, sl]
                return 0

            lax.fori_loop(0, sb, body, 0)
            pltpu.sync_copy(ra_v, out_hbm.at[pl.ds(off, sb)])

    return k(ys, pa, pb)


def kernel(x, gate_w, gate_b, w1, b1, w2, b2):
    bq, sq, dq = x.shape
    n = bq * sq
    e, _, ff = w1.shape
    topk = 2
    bm = 512
    npad = ((n * topk + e * (bm - 1)) // bm + 1) * bm

    xf = x.reshape(n, dq)
    posa, posb, wa, wb, counts, aux = _router(xf, gate_w, gate_b, topk, bm)
    pa = posa.reshape(n)
    pb = posb.reshape(n)

    ci = counts.reshape(e).astype(jnp.int32)
    pc = ((ci + bm - 1) // bm) * bm
    ends = jnp.cumsum(pc)
    tile_start = jnp.arange(npad // bm, dtype=jnp.int32) * bm
    gid = jnp.sum((tile_start[:, None] >= ends[None, :]).astype(jnp.int32),
                  axis=1).astype(jnp.int32)
    act = (tile_start < ends[-1]).astype(jnp.int32)

    xs, ws = _dispatch_sc(xf, pa, pb, wa, wb, npad)

    ys = _gffn(xs, ws, gid, act, w1, b1, w2, b2, bm)

    out = _combine_sc(ys, pa, pb, n, dq)
    return out.reshape(bq, sq, dq), aux


# speedup vs baseline: 5.5557x; 1.0004x over previous
import functools

import jax
import jax.numpy as jnp
from jax import lax
from jax.experimental import pallas as pl
from jax.experimental.pallas import tpu as pltpu
from jax.experimental.pallas import tpu_sc as plsc


_SQRT_HALF = 0.7071067811865476


def _shift_down(a, d):
    return jnp.concatenate([jnp.zeros((d, a.shape[1]), a.dtype), a[:-d]], axis=0)


def _router_kernel(x_ref, gw_ref, gb_ref, posa_ref, posb_ref, wa_ref,
                   wb_ref, counts_ref, aux_ref, *, topk, bm):
    n = x_ref.shape[0]
    e = gw_ref.shape[1]
    logits = jnp.dot(x_ref[...], gw_ref[...],
                     preferred_element_type=jnp.float32) + gb_ref[...]
    m = jnp.max(logits, axis=-1, keepdims=True)
    ex = jnp.exp(logits - m)
    p = ex / jnp.sum(ex, axis=-1, keepdims=True)

    lane = jax.lax.broadcasted_iota(jnp.int32, (n, e), 1)
    v1 = jnp.max(p, axis=-1, keepdims=True)
    l1 = jnp.min(jnp.where(p == v1, lane, e), axis=-1, keepdims=True)
    sel1 = (lane == l1)
    p2 = jnp.where(sel1, -1.0, p)
    v2 = jnp.max(p2, axis=-1, keepdims=True)
    l2 = jnp.min(jnp.where(p2 == v2, lane, e), axis=-1, keepdims=True)
    sel2 = (lane == l2)

    den = v1 + v2
    wa_ref[...] = jnp.broadcast_to(v1 / den, wa_ref.shape)
    wb_ref[...] = jnp.broadcast_to(v2 / den, wb_ref.shape)

    c1 = sel1.astype(jnp.float32)
    c2 = sel2.astype(jnp.float32)
    d = 1
    while d < n:
        c1 = c1 + _shift_down(c1, d)
        c2 = c2 + _shift_down(c2, d)
        d *= 2
    tot1 = c1[n - 1:n, :]
    tot2 = c2[n - 1:n, :]
    counts = tot1 + tot2
    counts_ref[...] = counts

    pc = jnp.ceil(counts / bm) * bm
    inc = pc
    d = 1
    while d < e:
        inc = inc + jnp.concatenate(
            [jnp.zeros((1, d), jnp.float32), inc[:, :-d]], axis=1)
        d *= 2
    base = inc - pc

    posa_f = base + c1 - 1.0
    posb_f = base + tot1 + c2 - 1.0
    posa_ref[...] = jnp.sum(
        jnp.where(sel1, posa_f, 0.0), axis=1, keepdims=True).astype(jnp.int32)
    posb_ref[...] = jnp.sum(
        jnp.where(sel2, posb_f, 0.0), axis=1, keepdims=True).astype(jnp.int32)

    routing = jnp.mean(p, axis=0, keepdims=True)
    frac = counts / (n * topk)
    aux = e * jnp.sum(frac * routing)
    aux_ref[...] = jnp.broadcast_to(aux, (1, 1))


def _router(xf, gate_w, gate_b, topk, bm):
    n, d = xf.shape
    e = gate_w.shape[1]
    outs = pl.pallas_call(
        functools.partial(_router_kernel, topk=topk, bm=bm),
        out_shape=(
            jax.ShapeDtypeStruct((n, 1), jnp.int32),
            jax.ShapeDtypeStruct((n, 1), jnp.int32),
            jax.ShapeDtypeStruct((n, 128), jnp.float32),
            jax.ShapeDtypeStruct((n, 128), jnp.float32),
            jax.ShapeDtypeStruct((1, e), jnp.float32),
            jax.ShapeDtypeStruct((1, 1), jnp.float32),
        ),
    )(xf, gate_w, gate_b.reshape(1, e))
    posa, posb, wa, wb, counts, aux = outs
    return posa, posb, wa, wb, counts, aux[0, 0]


def _gffn_kernel(gid_ref, act_ref, xs_ref, ws_ref, w1_ref, b1_ref, w2_ref,
                 b2_ref, out_ref):
    t = pl.program_id(0)
    active = act_ref[t] == 1

    @pl.when(active)
    def _():
        x = xs_ref[...].astype(jnp.bfloat16)
        h = jnp.dot(x, w1_ref[0].astype(jnp.bfloat16),
                    preferred_element_type=jnp.float32) + b1_ref[0]
        h = 0.5 * h * (1.0 + jax.lax.erf(h * _SQRT_HALF))
        y = jnp.dot(h.astype(jnp.bfloat16), w2_ref[0].astype(jnp.bfloat16),
                    preferred_element_type=jnp.float32)
        out_ref[...] = (y + b2_ref[0]) * ws_ref[:, :1]


def _gffn(xs, ws, gid, act, w1, b1, w2, b2, bm):
    npad = xs.shape[0]
    e, d, ff = w1.shape
    t_tiles = npad // bm
    grid_spec = pltpu.PrefetchScalarGridSpec(
        num_scalar_prefetch=2,
        grid=(t_tiles,),
        in_specs=[
            pl.BlockSpec((bm, d), lambda t, gid, act: (t, 0)),
            pl.BlockSpec((bm, 128), lambda t, gid, act: (t, 0)),
            pl.BlockSpec((1, d, ff), lambda t, gid, act: (gid[t], 0, 0)),
            pl.BlockSpec((1, 1, ff), lambda t, gid, act: (gid[t], 0, 0)),
            pl.BlockSpec((1, ff, d), lambda t, gid, act: (gid[t], 0, 0)),
            pl.BlockSpec((1, 1, d), lambda t, gid, act: (gid[t], 0, 0)),
        ],
        out_specs=pl.BlockSpec((bm, d), lambda t, gid, act: (t, 0)),
    )
    return pl.pallas_call(
        _gffn_kernel,
        grid_spec=grid_spec,
        out_shape=jax.ShapeDtypeStruct((npad, d), jnp.float32),
    )(gid, act, xs, ws, w1, b1.reshape(e, 1, ff), w2, b2.reshape(e, 1, d))


def _dispatch_sc(xf, pa, pb, wa, wb, npad):
    n, d = xf.shape
    info = plsc.get_sparse_core_info()
    nw = info.num_cores * info.num_subcores
    per_w = n // nw
    mesh = plsc.VectorSubcoreMesh(core_axis_name="c", subcore_axis_name="s")

    @functools.partial(
        pl.kernel, mesh=mesh,
        out_type=(
            jax.ShapeDtypeStruct((npad, d), jnp.float32),
            jax.ShapeDtypeStruct((npad, 128), jnp.float32),
        ),
        scratch_types=[
            pltpu.VMEM((64,), jnp.int32),
            pltpu.VMEM((64,), jnp.int32),
            pltpu.VMEM((64, d), jnp.float32),
            pltpu.VMEM((64, 128), jnp.float32),
            pltpu.VMEM((64, 128), jnp.float32),
            pltpu.SemaphoreType.DMA,
        ],
    )
    def k(x_hbm, pa_hbm, pb_hbm, wa_hbm, wb_hbm, xs_hbm, ws_hbm,
          ia_v, ib_v, rows_v, wav_v, wbv_v, sem):
        wid = lax.axis_index("s") * info.num_cores + lax.axis_index("c")
        for blk in range(per_w // 64):
            base = wid * per_w + blk * 64
            pltpu.sync_copy(x_hbm.at[pl.ds(base, 64)], rows_v)
            pltpu.sync_copy(pa_hbm.at[pl.ds(base, 64)], ia_v)
            pltpu.sync_copy(pb_hbm.at[pl.ds(base, 64)], ib_v)
            pltpu.sync_copy(wa_hbm.at[pl.ds(base, 64)], wav_v)
            pltpu.sync_copy(wb_hbm.at[pl.ds(base, 64)], wbv_v)
            c1 = pltpu.async_copy(rows_v, xs_hbm.at[ia_v], sem)
            c2 = pltpu.async_copy(rows_v, xs_hbm.at[ib_v], sem)
            c3 = pltpu.async_copy(wav_v, ws_hbm.at[ia_v], sem)
            c4 = pltpu.async_copy(wbv_v, ws_hbm.at[ib_v], sem)
            c1.wait(); c2.wait(); c3.wait(); c4.wait()

    return k(xf, pa, pb, wa, wb)


def _combine_sc(ys, pa, pb, n, d):
    info = plsc.get_sparse_core_info()
    nw = info.num_cores * info.num_subcores
    per_w = n // nw
    sb = 64
    nlane = info.num_lanes
    mesh = plsc.VectorSubcoreMesh(core_axis_name="c", subcore_axis_name="s")

    @functools.partial(
        pl.kernel, mesh=mesh,
        out_type=jax.ShapeDtypeStruct((n, d), jnp.float32),
        scratch_types=[
            pltpu.VMEM((sb,), jnp.int32),
            pltpu.VMEM((sb,), jnp.int32),
            pltpu.VMEM((sb, d), jnp.float32),
            pltpu.VMEM((sb, d), jnp.float32),
            pltpu.SemaphoreType.DMA,
        ],
    )
    def k(ys_hbm, pa_hbm, pb_hbm, out_hbm, ia_v, ib_v, ra_v, rb_v, sem):
        wid = lax.axis_index("s") * info.num_cores + lax.axis_index("c")
        base = wid * per_w
        for blk in range(per_w // sb):
            off = base + blk * sb
            pltpu.sync_copy(pa_hbm.at[pl.ds(off, sb)], ia_v)
            pltpu.sync_copy(pb_hbm.at[pl.ds(off, sb)], ib_v)
            g1 = pltpu.async_copy(ys_hbm.at[ia_v], ra_v, sem)
            g2 = pltpu.async_copy(ys_hbm.at[ib_v], rb_v, sem)
            g1.wait(); g2.wait()

            def body(i, _):
                for j in range(d // nlane):
                    sl = pl.ds(j * nlane, nlane)
                    ra_v[i, sl] = ra_v[i, sl] + rb_v[i, sl]
                return 0

            lax.fori_loop(0, sb, body, 0)
            pltpu.sync_copy(ra_v, out_hbm.at[pl.ds(off, sb)])

    return k(ys, pa, pb)


def kernel(x, gate_w, gate_b, w1, b1, w2, b2):
    bq, sq, dq = x.shape
    n = bq * sq
    e, _, ff = w1.shape
    topk = 2
    bm = 1024
    npad = ((n * topk + e * (bm - 1)) // bm + 1) * bm

    xf = x.reshape(n, dq)
    posa, posb, wa, wb, counts, aux = _router(xf, gate_w, gate_b, topk, bm)
    pa = posa.reshape(n)
    pb = posb.reshape(n)

    ci = counts.reshape(e).astype(jnp.int32)
    pc = ((ci + bm - 1) // bm) * bm
    ends = jnp.cumsum(pc)
    tile_start = jnp.arange(npad // bm, dtype=jnp.int32) * bm
    gid = jnp.sum((tile_start[:, None] >= ends[None, :]).astype(jnp.int32),
                  axis=1).astype(jnp.int32)
    act = (tile_start < ends[-1]).astype(jnp.int32)

    xs, ws = _dispatch_sc(xf, pa, pb, wa, wb, npad)

    ys = _gffn(xs, ws, gid, act, w1, b1, w2, b2, bm)

    out = _combine_sc(ys, pa, pb, n, dq)
    return out.reshape(bq, sq, dq), aux


# speedup vs baseline: 5.5573x; 1.0003x over previous
import functools

import jax
import jax.numpy as jnp
from jax import lax
from jax.experimental import pallas as pl
from jax.experimental.pallas import tpu as pltpu
from jax.experimental.pallas import tpu_sc as plsc


_SQRT_HALF = 0.7071067811865476


def _shift_down(a, d):
    return jnp.concatenate([jnp.zeros((d, a.shape[1]), a.dtype), a[:-d]], axis=0)


def _router_kernel(x_ref, gw_ref, gb_ref, posa_ref, posb_ref, wa_ref,
                   wb_ref, gid_ref, act_ref, aux_ref, *, topk, bm):
    n = x_ref.shape[0]
    e = gw_ref.shape[1]
    logits = jnp.dot(x_ref[...], gw_ref[...],
                     preferred_element_type=jnp.float32) + gb_ref[...]
    m = jnp.max(logits, axis=-1, keepdims=True)
    ex = jnp.exp(logits - m)
    p = ex / jnp.sum(ex, axis=-1, keepdims=True)

    lane = jax.lax.broadcasted_iota(jnp.int32, (n, e), 1)
    v1 = jnp.max(p, axis=-1, keepdims=True)
    l1 = jnp.min(jnp.where(p == v1, lane, e), axis=-1, keepdims=True)
    sel1 = (lane == l1)
    p2 = jnp.where(sel1, -1.0, p)
    v2 = jnp.max(p2, axis=-1, keepdims=True)
    l2 = jnp.min(jnp.where(p2 == v2, lane, e), axis=-1, keepdims=True)
    sel2 = (lane == l2)

    den = v1 + v2
    wa_ref[...] = jnp.broadcast_to(v1 / den, wa_ref.shape)
    wb_ref[...] = jnp.broadcast_to(v2 / den, wb_ref.shape)

    c1 = sel1.astype(jnp.float32)
    c2 = sel2.astype(jnp.float32)
    d = 1
    while d < n:
        c1 = c1 + _shift_down(c1, d)
        c2 = c2 + _shift_down(c2, d)
        d *= 2
    tot1 = c1[n - 1:n, :]
    tot2 = c2[n - 1:n, :]
    counts = tot1 + tot2

    pc = jnp.ceil(counts / bm) * bm
    inc = pc
    d = 1
    while d < e:
        inc = inc + jnp.concatenate(
            [jnp.zeros((1, d), jnp.float32), inc[:, :-d]], axis=1)
        d *= 2
    base = inc - pc

    t_tiles = gid_ref.shape[1]
    lane8 = jax.lax.broadcasted_iota(jnp.int32, (e, e), 0)
    eye = (lane8 == jax.lax.broadcasted_iota(jnp.int32, (e, e), 1))
    ends_t = jax.lax.dot_general(eye.astype(jnp.float32), inc,
                                 (((1,), (1,)), ((), ())),
                                 preferred_element_type=jnp.float32)
    tb = jax.lax.broadcasted_iota(jnp.int32, (e, t_tiles), 1).astype(jnp.float32) * bm
    gid_ref[...] = jnp.sum((tb >= ends_t).astype(jnp.int32), axis=0,
                           keepdims=True)
    maxend = jnp.max(inc)
    tb1 = jax.lax.broadcasted_iota(jnp.int32, (1, t_tiles), 1).astype(jnp.float32) * bm
    act_ref[...] = (tb1 < maxend).astype(jnp.int32)

    posa_f = base + c1 - 1.0
    posb_f = base + tot1 + c2 - 1.0
    posa_ref[...] = jnp.sum(
        jnp.where(sel1, posa_f, 0.0), axis=1, keepdims=True).astype(jnp.int32)
    posb_ref[...] = jnp.sum(
        jnp.where(sel2, posb_f, 0.0), axis=1, keepdims=True).astype(jnp.int32)

    routing = jnp.mean(p, axis=0, keepdims=True)
    frac = counts / (n * topk)
    aux = e * jnp.sum(frac * routing)
    aux_ref[...] = jnp.broadcast_to(aux, (1, 1))


def _router(xf, gate_w, gate_b, topk, bm, t_tiles):
    n, d = xf.shape
    e = gate_w.shape[1]
    outs = pl.pallas_call(
        functools.partial(_router_kernel, topk=topk, bm=bm),
        out_shape=(
            jax.ShapeDtypeStruct((n, 1), jnp.int32),
            jax.ShapeDtypeStruct((n, 1), jnp.int32),
            jax.ShapeDtypeStruct((n, 128), jnp.float32),
            jax.ShapeDtypeStruct((n, 128), jnp.float32),
            jax.ShapeDtypeStruct((1, t_tiles), jnp.int32),
            jax.ShapeDtypeStruct((1, t_tiles), jnp.int32),
            jax.ShapeDtypeStruct((1, 1), jnp.float32),
        ),
    )(xf, gate_w, gate_b.reshape(1, e))
    posa, posb, wa, wb, gid, act, aux = outs
    return posa, posb, wa, wb, gid, act, aux[0, 0]


def _gffn_kernel(gid_ref, act_ref, xs_ref, ws_ref, w1_ref, b1_ref, w2_ref,
                 b2_ref, out_ref):
    t = pl.program_id(0)
    active = act_ref[0, t] == 1

    @pl.when(active)
    def _():
        x = xs_ref[...].astype(jnp.bfloat16)
        h = jnp.dot(x, w1_ref[0].astype(jnp.bfloat16),
                    preferred_element_type=jnp.float32) + b1_ref[0]
        h = 0.5 * h * (1.0 + jax.lax.erf(h * _SQRT_HALF))
        y = jnp.dot(h.astype(jnp.bfloat16), w2_ref[0].astype(jnp.bfloat16),
                    preferred_element_type=jnp.float32)
        out_ref[...] = (y + b2_ref[0]) * ws_ref[:, :1]


def _gffn(xs, ws, gid, act, w1, b1, w2, b2, bm):
    npad = xs.shape[0]
    e, d, ff = w1.shape
    t_tiles = npad // bm
    grid_spec = pltpu.PrefetchScalarGridSpec(
        num_scalar_prefetch=2,
        grid=(t_tiles,),
        in_specs=[
            pl.BlockSpec((bm, d), lambda t, gid, act: (t, 0)),
            pl.BlockSpec((bm, 128), lambda t, gid, act: (t, 0)),
            pl.BlockSpec((1, d, ff), lambda t, gid, act: (gid[0, t], 0, 0)),
            pl.BlockSpec((1, 1, ff), lambda t, gid, act: (gid[0, t], 0, 0)),
            pl.BlockSpec((1, ff, d), lambda t, gid, act: (gid[0, t], 0, 0)),
            pl.BlockSpec((1, 1, d), lambda t, gid, act: (gid[0, t], 0, 0)),
        ],
        out_specs=pl.BlockSpec((bm, d), lambda t, gid, act: (t, 0)),
    )
    return pl.pallas_call(
        _gffn_kernel,
        grid_spec=grid_spec,
        out_shape=jax.ShapeDtypeStruct((npad, d), jnp.float32),
    )(gid, act, xs, ws, w1, b1.reshape(e, 1, ff), w2, b2.reshape(e, 1, d))


def _dispatch_sc(xf, pa, pb, wa, wb, npad):
    n, d = xf.shape
    info = plsc.get_sparse_core_info()
    nw = info.num_cores * info.num_subcores
    per_w = n // nw
    mesh = plsc.VectorSubcoreMesh(core_axis_name="c", subcore_axis_name="s")

    @functools.partial(
        pl.kernel, mesh=mesh,
        out_type=(
            jax.ShapeDtypeStruct((npad, d), jnp.float32),
            jax.ShapeDtypeStruct((npad, 128), jnp.float32),
        ),
        scratch_types=[
            pltpu.VMEM((64,), jnp.int32),
            pltpu.VMEM((64,), jnp.int32),
            pltpu.VMEM((64, d), jnp.float32),
            pltpu.VMEM((64, 128), jnp.float32),
            pltpu.VMEM((64, 128), jnp.float32),
            pltpu.SemaphoreType.DMA,
        ],
    )
    def k(x_hbm, pa_hbm, pb_hbm, wa_hbm, wb_hbm, xs_hbm, ws_hbm,
          ia_v, ib_v, rows_v, wav_v, wbv_v, sem):
        wid = lax.axis_index("s") * info.num_cores + lax.axis_index("c")
        for blk in range(per_w // 64):
            base = wid * per_w + blk * 64
            pltpu.sync_copy(x_hbm.at[pl.ds(base, 64)], rows_v)
            pltpu.sync_copy(pa_hbm.at[pl.ds(base, 64)], ia_v)
            pltpu.sync_copy(pb_hbm.at[pl.ds(base, 64)], ib_v)
            pltpu.sync_copy(wa_hbm.at[pl.ds(base, 64)], wav_v)
            pltpu.sync_copy(wb_hbm.at[pl.ds(base, 64)], wbv_v)
            c1 = pltpu.async_copy(rows_v, xs_hbm.at[ia_v], sem)
            c2 = pltpu.async_copy(rows_v, xs_hbm.at[ib_v], sem)
            c3 = pltpu.async_copy(wav_v, ws_hbm.at[ia_v], sem)
            c4 = pltpu.async_copy(wbv_v, ws_hbm.at[ib_v], sem)
            c1.wait(); c2.wait(); c3.wait(); c4.wait()

    return k(xf, pa, pb, wa, wb)


def _combine_sc(ys, pa, pb, n, d):
    info = plsc.get_sparse_core_info()
    nw = info.num_cores * info.num_subcores
    per_w = n // nw
    sb = 64
    nlane = info.num_lanes
    mesh = plsc.VectorSubcoreMesh(core_axis_name="c", subcore_axis_name="s")

    @functools.partial(
        pl.kernel, mesh=mesh,
        out_type=jax.ShapeDtypeStruct((n, d), jnp.float32),
        scratch_types=[
            pltpu.VMEM((sb,), jnp.int32),
            pltpu.VMEM((sb,), jnp.int32),
            pltpu.VMEM((sb, d), jnp.float32),
            pltpu.VMEM((sb, d), jnp.float32),
            pltpu.SemaphoreType.DMA,
        ],
    )
    def k(ys_hbm, pa_hbm, pb_hbm, out_hbm, ia_v, ib_v, ra_v, rb_v, sem):
        wid = lax.axis_index("s") * info.num_cores + lax.axis_index("c")
        base = wid * per_w
        for blk in range(per_w // sb):
            off = base + blk * sb
            pltpu.sync_copy(pa_hbm.at[pl.ds(off, sb)], ia_v)
            pltpu.sync_copy(pb_hbm.at[pl.ds(off, sb)], ib_v)
            g1 = pltpu.async_copy(ys_hbm.at[ia_v], ra_v, sem)
            g2 = pltpu.async_copy(ys_hbm.at[ib_v], rb_v, sem)
            g1.wait(); g2.wait()

            def body(i, _):
                for j in range(d // nlane):
                    sl = pl.ds(j * nlane, nlane)
                    ra_v[i, sl] = ra_v[i, sl] + rb_v[i, sl]
                return 0

            lax.fori_loop(0, sb, body, 0)
            pltpu.sync_copy(ra_v, out_hbm.at[pl.ds(off, sb)])

    return k(ys, pa, pb)


def kernel(x, gate_w, gate_b, w1, b1, w2, b2):
    bq, sq, dq = x.shape
    n = bq * sq
    e, _, ff = w1.shape
    topk = 2
    bm = 512
    npad = ((n * topk + e * (bm - 1)) // bm + 1) * bm

    xf = x.reshape(n, dq)
    posa, posb, wa, wb, gid, act, aux = _router(
        xf, gate_w, gate_b, topk, bm, npad // bm)
    pa = posa.reshape(n)
    pb = posb.reshape(n)

    xs, ws = _dispatch_sc(xf, pa, pb, wa, wb, npad)

    ys = _gffn(xs, ws, gid, act, w1, b1, w2, b2, bm)

    out = _combine_sc(ys, pa, pb, n, dq)
    return out.reshape(bq, sq, dq), aux
